# Initial kernel scaffold; baseline (speedup 1.0000x reference)
import jax
import jax.numpy as jnp
from jax.experimental import pallas as pl


def kernel(data, spelling_table, char_emb, W_ih, W_hh, b_ih, b_hh):
    raise NotImplementedError("write your pallas kernel here")


# speedup vs baseline: 5.2270x; 5.2270x over previous
import jax
import jax.numpy as jnp
from jax import lax
from jax.experimental import pallas as pl
from jax.experimental.pallas import tpu as pltpu

MAX_LEN = 20
CEMB = 32
H = 128
BT = 512


def _gru_body(spell_ref, len_ref, cemb_ref, wih_ref, whh_ref, bih_ref,
              bhh_ref, out_ref, h_ref):
    lengths = len_ref[...]
    maxlen = jnp.max(lengths)
    cemb = cemb_ref[...]
    wih = wih_ref[...]
    whh = whh_ref[...]
    bih = bih_ref[...]
    bhh = bhh_ref[...]
    bt = out_ref.shape[0]
    iota = lax.broadcasted_iota(jnp.int32, (bt, 128), 1)
    h_ref[...] = jnp.zeros((bt, H), jnp.float32)

    for t in range(MAX_LEN):
        @pl.when(t < maxlen)
        def _step():
            h = h_ref[...]
            c = spell_ref[:, t:t + 1]
            oh = (c == iota).astype(jnp.float32)
            x = jnp.dot(oh, cemb, preferred_element_type=jnp.float32)
            gi = jnp.dot(x, wih, preferred_element_type=jnp.float32) + bih
            gh = jnp.dot(h, whh, preferred_element_type=jnp.float32) + bhh
            r = jax.nn.sigmoid(gi[:, :H] + gh[:, :H])
            z = jax.nn.sigmoid(gi[:, H:2 * H] + gh[:, H:2 * H])
            n = jnp.tanh(gi[:, 2 * H:] + r * gh[:, 2 * H:])
            hn = (1.0 - z) * n + z * h
            h_ref[...] = jnp.where(t < lengths, hn, h)

    out_ref[...] = h_ref[...]


def _gru_sorted(s_spell, s_len, char_emb, wihT, whhT, bih2, bhh2):
    B = s_spell.shape[0]
    grid = (B // BT,)
    return pl.pallas_call(
        _gru_body,
        grid=grid,
        in_specs=[
            pl.BlockSpec((BT, MAX_LEN), lambda i: (i, 0)),
            pl.BlockSpec((BT, 1), lambda i: (i, 0)),
            pl.BlockSpec((128, CEMB), lambda i: (0, 0)),
            pl.BlockSpec((CEMB, 3 * H), lambda i: (0, 0)),
            pl.BlockSpec((H, 3 * H), lambda i: (0, 0)),
            pl.BlockSpec((1, 3 * H), lambda i: (0, 0)),
            pl.BlockSpec((1, 3 * H), lambda i: (0, 0)),
        ],
        out_specs=pl.BlockSpec((BT, H), lambda i: (i, 0)),
        out_shape=jax.ShapeDtypeStruct((B, H), jnp.float32),
        scratch_shapes=[pltpu.VMEM((BT, H), jnp.float32)],
    )(s_spell, s_len, char_emb, wihT, whhT, bih2, bhh2)


def kernel(data, spelling_table, char_emb, W_ih, W_hh, b_ih, b_hh):
    rows = spelling_table[data]
    spelling = rows[:, :MAX_LEN]
    lengths = rows[:, MAX_LEN]
    order = jnp.argsort(-lengths)
    s_spell = spelling[order]
    s_len = lengths[order]
    h_sorted = _gru_sorted(s_spell, s_len[:, None], char_emb,
                           W_ih.T, W_hh.T, b_ih[None, :], b_hh[None, :])
    return h_sorted[order]
